# baseline (speedup 1.0000x reference)
import functools

import jax
import jax.numpy as jnp
from jax import lax
from jax.experimental import pallas as pl
from jax.experimental.pallas import tpu as pltpu
from jax.experimental.pallas import tpu_sc as plsc

B, S, D, H, L, K = 8, 1024, 64, 4, 2, 8192
DK = D // H
FF = 4 * D
N = B * S


def _rowsum(x):
    w = x.shape[1]
    if w > 128:
        acc = x[:, 0:128]
        for t in range(1, w // 128):
            acc = acc + x[:, t * 128:(t + 1) * 128]
    else:
        acc = x
    ng = acc.shape[1] // 8
    part = acc[:, 0:8]
    for j in range(1, ng):
        part = part + acc[:, j * 8:(j + 1) * 8]
    part = part[:, 0:4] + part[:, 4:8]
    part = part[:, 0:2] + part[:, 2:4]
    part = part[:, 0:1] + part[:, 1:2]
    return part


def _ln(x, g, b):
    m = _rowsum(x) / jnp.float32(x.shape[1])
    c = x - m
    v = _rowsum(c * c) / jnp.float32(x.shape[1])
    return c / jnp.sqrt(v + 1e-5) * g + b


def _softmax(s):
    m = jnp.max(s, axis=-1, keepdims=True)
    e = jnp.exp(s - m)
    return e / _rowsum(e)


def _encoder_body(x_ref, wq_ref, bq_ref, wk_ref, bk_ref, wv_ref, bv_ref,
                  wo_ref, bo_ref, ln1g_ref, ln1b_ref, w1_ref, b1_ref,
                  w2_ref, b2_ref, ln2g_ref, ln2b_ref, out_ref):
    h = x_ref[0]
    scale = jnp.float32(1.0) / jnp.sqrt(jnp.float32(DK))
    for l in range(L):
        q = jnp.dot(h, wq_ref[l]) + bq_ref[l]
        k = jnp.dot(h, wk_ref[l]) + bk_ref[l]
        v = jnp.dot(h, wv_ref[l]) + bv_ref[l]
        ctx_heads = []
        for hh in range(H):
            sl = slice(hh * DK, (hh + 1) * DK)
            s = lax.dot_general(q[:, sl], k[:, sl],
                                (((1,), (1,)), ((), ()))) * scale
            aw = _softmax(s)
            ctx_heads.append(jnp.dot(aw, v[:, sl]))
        ctx = jnp.concatenate(ctx_heads, axis=1)
        attn = jnp.dot(ctx, wo_ref[l]) + bo_ref[l]
        h = _ln(h + attn, ln1g_ref[l], ln1b_ref[l])
        ffp = jnp.dot(h, w1_ref[l]) + b1_ref[l]
        ff = ffp * 0.5 * (1.0 + lax.erf(ffp * jnp.float32(0.7071067811865476)))
        ff = jnp.dot(ff, w2_ref[l]) + b2_ref[l]
        h = _ln(h + ff, ln2g_ref[l], ln2b_ref[l])
    out_ref[0] = h


def _run_encoder(x, wq, bq, wk, bk, wv, bv, wo, bo,
                 ln1_g, ln1_b, w1, b1, w2, b2, ln2_g, ln2_b):
    full = lambda *shape: pl.BlockSpec(shape, lambda i: (0,) * len(shape))
    return pl.pallas_call(
        _encoder_body,
        grid=(B,),
        in_specs=[
            pl.BlockSpec((1, S, D), lambda i: (i, 0, 0)),
            full(L, D, D), full(L, D), full(L, D, D), full(L, D),
            full(L, D, D), full(L, D), full(L, D, D), full(L, D),
            full(L, D), full(L, D),
            full(L, D, FF), full(L, FF), full(L, FF, D), full(L, D),
            full(L, D), full(L, D),
        ],
        out_specs=pl.BlockSpec((1, S, D), lambda i: (i, 0, 0)),
        out_shape=jax.ShapeDtypeStruct((B, S, D), jnp.float32),
        compiler_params=pltpu.CompilerParams(
            dimension_semantics=("arbitrary",)),
    )(x, wq, bq, wk, bk, wv, bv, wo, bo, ln1_g, ln1_b, w1, b1, w2, b2,
      ln2_g, ln2_b)


_RT = 1024
_KT = 1024


_CC = 2048


def _vq_body(flat_ref, cb_ref, idx_ref, cnt_ref):
    rows = flat_ref[...]
    x2 = _rowsum(rows * rows)
    best = jnp.full((_RT,), jnp.inf, jnp.float32)
    barg = jnp.zeros((_RT,), jnp.int32)
    for c in range(K // _CC):
        cmin = jnp.full((_RT,), jnp.inf, jnp.float32)
        carg = jnp.zeros((_RT,), jnp.int32)
        for t in range(_CC // _KT):
            base = c * _CC + t * _KT
            cb_t = cb_ref[pl.ds(base, _KT), :]
            c2 = _rowsum(cb_t * cb_t)[:, 0]
            mm = lax.dot_general(rows, cb_t, (((1,), (1,)), ((), ())))
            dist = (x2 + c2[None, :]) - 2.0 * mm
            tmin = jnp.min(dist, axis=1)
            li = lax.broadcasted_iota(jnp.int32, (_RT, _KT), 1)
            targ = jnp.min(
                jnp.where(dist == tmin[:, None], li, jnp.int32(2**30)),
                axis=1) + base
            upd = tmin < cmin
            carg = jnp.where(upd, targ, carg)
            cmin = jnp.where(upd, tmin, cmin)
        bq = best.astype(jnp.bfloat16).astype(jnp.float32)
        upd = cmin < bq
        barg = jnp.where(upd, carg, barg)
        best = jnp.where(upd, cmin, best)
    idx_ref[0, 0] = barg
    for t in range(K // _KT):
        ci = lax.broadcasted_iota(jnp.int32, (_RT, _KT), 1) + t * _KT
        eq = (barg[:, None] == ci).astype(jnp.float32)
        cnt_ref[0, 0, pl.ds(t * _KT, _KT)] = jnp.sum(eq, axis=0)


def _run_vq(flat, codebook):
    idx3, counts = pl.pallas_call(
        _vq_body,
        grid=(N // _RT,),
        in_specs=[
            pl.BlockSpec((_RT, D), lambda i: (i, 0)),
            pl.BlockSpec((K, D), lambda i: (0, 0)),
        ],
        out_specs=[
            pl.BlockSpec((1, 1, _RT), lambda i: (i, 0, 0)),
            pl.BlockSpec((1, 1, K), lambda i: (i, 0, 0)),
        ],
        out_shape=[
            jax.ShapeDtypeStruct((N // _RT, 1, _RT), jnp.int32),
            jax.ShapeDtypeStruct((N // _RT, 1, K), jnp.float32),
        ],
        compiler_params=pltpu.CompilerParams(
            dimension_semantics=("arbitrary",)),
    )(flat, codebook)
    return idx3.reshape(N), counts.reshape(N // _RT, K)


_DP = 128


def _make_sc_gather():
    info = plsc.get_sparse_core_info()
    nw = info.num_cores * info.num_subcores
    b_per_w = N // nw
    chunks = -(-b_per_w // 128)
    chunk = b_per_w // chunks
    mesh = plsc.VectorSubcoreMesh(core_axis_name="c", subcore_axis_name="s")

    @functools.partial(
        pl.kernel, mesh=mesh,
        out_type=jax.ShapeDtypeStruct((N, _DP), jnp.float32),
        scratch_types=[
            pltpu.VMEM((chunks, chunk), jnp.int32),
            pltpu.VMEM((b_per_w, _DP), jnp.float32),
            pltpu.SemaphoreType.DMA,
        ],
    )
    def sc_gather(cb_hbm, idx_hbm, out_hbm, idx_v, rows_v, sem):
        wid = lax.axis_index("s") * info.num_cores + lax.axis_index("c")
        base = wid * b_per_w
        for j in range(chunks):
            pltpu.sync_copy(idx_hbm.at[pl.ds(base + j * chunk, chunk)],
                            idx_v.at[j])
        descs = [
            pltpu.async_copy(cb_hbm.at[idx_v.at[j]],
                             rows_v.at[pl.ds(j * chunk, chunk)], sem)
            for j in range(chunks)
        ]
        for dsc in descs:
            dsc.wait()
        pltpu.sync_copy(rows_v, out_hbm.at[pl.ds(base, b_per_w)])

    return sc_gather


def _epilogue_body(h_ref, q_ref, cnt_ref, qst_ref, loss_ref, perp_ref):
    hh = h_ref[...]
    qq = q_ref[:, :D]
    d = qq - hh
    qst_ref[...] = hh + d
    m = jnp.mean(d * d)
    loss_ref[...] = jnp.broadcast_to(m + 0.25 * m, (1, 1))
    counts = jnp.sum(cnt_ref[...], axis=0)
    avg = counts / jnp.float32(N)
    perp = jnp.exp(-jnp.sum(avg * jnp.log(avg + 1e-10)))
    perp_ref[...] = jnp.broadcast_to(perp, (1, 1))


def _run_epilogue(flat, quant, counts):
    return pl.pallas_call(
        _epilogue_body,
        out_shape=[
            jax.ShapeDtypeStruct((N, D), jnp.float32),
            jax.ShapeDtypeStruct((1, 1), jnp.float32),
            jax.ShapeDtypeStruct((1, 1), jnp.float32),
        ],
    )(flat, quant, counts)


def kernel(x, wq, bq, wk, bk, wv, bv, wo, bo, ln1_g, ln1_b, w1, b1, w2, b2,
           ln2_g, ln2_b, codebook):
    h = _run_encoder(x, wq, bq, wk, bk, wv, bv, wo, bo,
                     ln1_g, ln1_b, w1, b1, w2, b2, ln2_g, ln2_b)
    flat = h.reshape(N, D)
    idx, counts = _run_vq(flat, codebook)
    cb_pad = jnp.pad(codebook, ((0, 0), (0, _DP - D)))
    quant = _make_sc_gather()(cb_pad, idx)
    qst, loss, perp = _run_epilogue(flat, quant, counts)
    return (qst.reshape(B, S, D), loss.reshape(()), perp.reshape(()), idx)
